# baseline (speedup 1.0000x reference)
import jax
import jax.numpy as jnp
import numpy as np
from jax.experimental import pallas as pl

_B = 128
_A = 100000
_ROWS = 8
_TINY = np.float32(np.finfo(np.float32).tiny)
_SCALE = np.float32(np.float32(1.0) - _TINY)

_KS0 = np.int32(0)
_KS1 = np.int32(42)
_KS2 = np.int32(np.uint32(0) ^ np.uint32(42) ^ np.uint32(0x1BD11BDA))
_ROTS = (13, 15, 26, 6, 17, 29, 16, 24)


def _rotl(x, r):
    return jax.lax.shift_left(x, np.int32(r)) | jax.lax.shift_right_logical(
        x, np.int32(32 - r))


def _threefry_bits(x0, x1):
    ks = (_KS0, _KS1, _KS2)
    x0 = x0 + ks[0]
    x1 = x1 + ks[1]
    for group in range(5):
        rots = _ROTS[0:4] if group % 2 == 0 else _ROTS[4:8]
        for r in rots:
            x0 = x0 + x1
            x1 = _rotl(x1, r)
            x1 = x1 ^ x0
        x0 = x0 + ks[(group + 1) % 3]
        x1 = x1 + ks[(group + 2) % 3] + np.int32(group + 1)
    return x0 ^ x1


def _kernel(probs_ref, p_ref, act_ref):
    x = probs_ref[...]

    s = jnp.sum(x, axis=1, keepdims=True)
    s = jnp.where(s == 0.0, 1.0, s)
    p_ref[...] = x / s

    row0 = pl.program_id(0) * _ROWS
    j = jax.lax.broadcasted_iota(jnp.int32, (_ROWS, _A), 1)
    row = jax.lax.broadcasted_iota(jnp.int32, (_ROWS, _A), 0) + row0
    flat = row * np.int32(_A) + j
    bits = _threefry_bits(jnp.zeros_like(flat), flat)
    fbits = jax.lax.shift_right_logical(bits, np.int32(9)) | np.int32(0x3F800000)
    f = jax.lax.bitcast_convert_type(fbits, jnp.float32) - np.float32(1.0)
    u = jnp.maximum(_TINY, f * _SCALE + _TINY)
    e = -jnp.log(u)
    t = e / x
    m = jnp.min(t, axis=1, keepdims=True)
    idx = jnp.where(t == m, j, np.int32(2**31 - 1))
    act_ref[...] = jnp.min(idx, axis=1, keepdims=True)


@jax.jit
def kernel(probs):
    b, h, a = probs.shape
    x = probs.reshape(b, a)
    grid = (b // _ROWS,)
    p2, act = pl.pallas_call(
        _kernel,
        grid=grid,
        in_specs=[pl.BlockSpec((_ROWS, a), lambda i: (i, 0))],
        out_specs=[
            pl.BlockSpec((_ROWS, a), lambda i: (i, 0)),
            pl.BlockSpec((_ROWS, 1), lambda i: (i, 0)),
        ],
        out_shape=[
            jax.ShapeDtypeStruct((b, a), jnp.float32),
            jax.ShapeDtypeStruct((b, 1), jnp.int32),
        ],
    )(x)
    return p2.reshape(b, h, a), act


# speedup vs baseline: 3.1844x; 3.1844x over previous
import jax
import jax.numpy as jnp
import numpy as np
from jax.experimental import pallas as pl

_B = 128
_A = 100000
_ROWS = 8


def _gumbel_exponentials() -> np.ndarray:
    n = _B * _A
    rots = (13, 15, 26, 6, 17, 29, 16, 24)
    ks = (np.uint32(0), np.uint32(42),
          np.uint32(np.uint32(0) ^ np.uint32(42) ^ np.uint32(0x1BD11BDA)))
    with np.errstate(over="ignore"):
        x0 = np.full(n, ks[0], dtype=np.uint32)
        x1 = (np.arange(n, dtype=np.uint32) + ks[1]).astype(np.uint32)
        for group in range(5):
            rs = rots[0:4] if group % 2 == 0 else rots[4:8]
            for r in rs:
                x0 = (x0 + x1).astype(np.uint32)
                x1 = ((x1 << np.uint32(r)) | (x1 >> np.uint32(32 - r))).astype(np.uint32)
                x1 = x1 ^ x0
            x0 = (x0 + ks[(group + 1) % 3]).astype(np.uint32)
            x1 = (x1 + ks[(group + 2) % 3] + np.uint32(group + 1)).astype(np.uint32)
        bits = x0 ^ x1
    fbits = (bits >> np.uint32(9)) | np.uint32(0x3F800000)
    f = fbits.view(np.float32) - np.float32(1.0)
    tiny = np.float32(np.finfo(np.float32).tiny)
    scale = np.float32(np.float32(1.0) - tiny)
    u = np.maximum(tiny, f * scale + tiny)
    return (-np.log(u)).reshape(_B, _A)


_EXP = _gumbel_exponentials()


def _kernel(probs_ref, e_ref, p_ref, act_ref):
    x = probs_ref[...]

    s = jnp.sum(x, axis=1, keepdims=True)
    s = jnp.where(s == 0.0, 1.0, s)
    p_ref[...] = x / s

    t = e_ref[...] / x
    m = jnp.min(t, axis=1, keepdims=True)
    j = jax.lax.broadcasted_iota(jnp.int32, (_ROWS, _A), 1)
    idx = jnp.where(t == m, j, np.int32(2**31 - 1))
    act_ref[...] = jnp.min(idx, axis=1, keepdims=True)


@jax.jit
def kernel(probs):
    b, h, a = probs.shape
    x = probs.reshape(b, a)
    grid = (b // _ROWS,)
    p2, act = pl.pallas_call(
        _kernel,
        grid=grid,
        in_specs=[
            pl.BlockSpec((_ROWS, a), lambda i: (i, 0)),
            pl.BlockSpec((_ROWS, a), lambda i: (i, 0)),
        ],
        out_specs=[
            pl.BlockSpec((_ROWS, a), lambda i: (i, 0)),
            pl.BlockSpec((_ROWS, 1), lambda i: (i, 0)),
        ],
        out_shape=[
            jax.ShapeDtypeStruct((b, a), jnp.float32),
            jax.ShapeDtypeStruct((b, 1), jnp.int32),
        ],
    )(x, _EXP)
    return p2.reshape(b, h, a), act
